# Initial kernel scaffold; baseline (speedup 1.0000x reference)
import jax
import jax.numpy as jnp
from jax.experimental import pallas as pl


def kernel(x, edge_index, W1, b1, W2, b2, W3, b3, Wo1, bo1, Wo2, bo2):
    raise NotImplementedError("write your pallas kernel here")


# speedup vs baseline: 8.0545x; 8.0545x over previous
import functools

import jax
import jax.numpy as jnp
from jax import lax
from jax.experimental import pallas as pl
from jax.experimental.pallas import tpu as pltpu
from jax.experimental.pallas import tpu_sc as plsc

N = 10000
E = 320000
D_IN = 128
D_HID = 256
D_OUT = 128
H = 128
LANE = 128
NT = 16
Np = 10240
RPT = Np // NT
EPT = E // NT
NCH = -(-EPT // LANE)
EPT_D = E // (2 * NT)
NCH_D = -(-EPT_D // LANE)
TR = 1000
GR = N // TR

_mesh = plsc.VectorSubcoreMesh(core_axis_name="c", subcore_axis_name="s")


@functools.partial(
    pl.kernel,
    out_type=jax.ShapeDtypeStruct((2 * Np, H), jnp.float32),
    mesh=_mesh,
    scratch_types=[
        pltpu.VMEM((2, LANE), jnp.int32),
        pltpu.VMEM((LANE, H), jnp.float32),
        pltpu.VMEM_SHARED((Np, H), jnp.float32),
        pltpu.SemaphoreType.DMA,
    ],
)
def _agg(hp_hbm, sd_hbm, out_hbm, sd_v, rows_v, acc_sh, sem):
    c = lax.axis_index("c")
    s = lax.axis_index("s")
    wid = c * NT + s
    base = c * Np + s * RPT
    pltpu.sync_copy(hp_hbm.at[pl.ds(base, RPT)], acc_sh.at[pl.ds(s * RPT, RPT)])
    plsc.subcore_barrier()

    def body(j, carry):
        pltpu.sync_copy(sd_hbm.at[wid].at[j], sd_v)
        pltpu.async_copy(hp_hbm.at[sd_v.at[0]], rows_v, sem).wait()
        pltpu.sync_copy(rows_v, acc_sh.at[sd_v.at[1]], add=True)
        return carry

    lax.fori_loop(0, NCH, body, 0)
    plsc.subcore_barrier()
    pltpu.sync_copy(acc_sh.at[pl.ds(s * RPT, RPT)], out_hbm.at[pl.ds(base, RPT)])


def _l1_body(x_ref, w_ref, degt_ref, hp_ref, dinv_ref):
    deg = degt_ref[:, :1]
    dinv = lax.rsqrt(deg)
    h = jnp.dot(x_ref[...], w_ref[...], preferred_element_type=jnp.float32)
    hp_ref[0] = h * dinv
    dinv_ref[...] = dinv


def _mid_body(agg_ref, dinv_ref, b_ref, w_ref, hp_ref):
    a = agg_ref[...]
    dinv = dinv_ref[...]
    b = b_ref[...]
    t0 = jnp.maximum(a[0] * dinv + b[:, :H], 0.0)
    t1 = jnp.maximum(a[1] * dinv + b[:, H:], 0.0)
    h = (jnp.dot(t0, w_ref[:H], preferred_element_type=jnp.float32)
         + jnp.dot(t1, w_ref[H:], preferred_element_type=jnp.float32))
    hp_ref[0] = h * dinv


def _fin_body(agg_ref, dinv_ref, b3_ref, wo1_ref, bo1_ref, wo2_ref, bo2_ref,
              out_ref):
    a = agg_ref[...]
    dinv = dinv_ref[...]
    b3 = b3_ref[...]
    t0 = jnp.maximum(a[0] * dinv + b3[:, :H], 0.0)
    t1 = jnp.maximum(a[1] * dinv + b3[:, H:], 0.0)
    u = (jnp.dot(t0, wo1_ref[:H], preferred_element_type=jnp.float32)
         + jnp.dot(t1, wo1_ref[H:], preferred_element_type=jnp.float32)
         + bo1_ref[...])
    out_ref[...] = (jnp.dot(u, wo2_ref[...], preferred_element_type=jnp.float32)
                    + bo2_ref[...])


def _layer1(x, W1, degt):
    return pl.pallas_call(
        _l1_body,
        grid=(GR, 2),
        in_specs=[
            pl.BlockSpec((TR, D_IN), lambda r, j: (r, 0)),
            pl.BlockSpec((D_IN, H), lambda r, j: (0, j)),
            pl.BlockSpec((TR, H), lambda r, j: (r, 0)),
        ],
        out_specs=[
            pl.BlockSpec((1, TR, H), lambda r, j: (j, r, 0)),
            pl.BlockSpec((TR, 1), lambda r, j: (r, 0)),
        ],
        out_shape=[
            jax.ShapeDtypeStruct((2, Np, H), jnp.float32),
            jax.ShapeDtypeStruct((N, 1), jnp.float32),
        ],
    )(x, W1, degt)


def _mid(agg, dinv, b, W):
    return pl.pallas_call(
        _mid_body,
        grid=(GR, 2),
        in_specs=[
            pl.BlockSpec((2, TR, H), lambda r, j: (0, r, 0)),
            pl.BlockSpec((TR, 1), lambda r, j: (r, 0)),
            pl.BlockSpec((1, D_HID), lambda r, j: (0, 0)),
            pl.BlockSpec((D_HID, H), lambda r, j: (0, j)),
        ],
        out_specs=pl.BlockSpec((1, TR, H), lambda r, j: (j, r, 0)),
        out_shape=jax.ShapeDtypeStruct((2, Np, H), jnp.float32),
    )(agg, dinv, b, W)


def _final(agg, dinv, b3, Wo1, bo1, Wo2, bo2):
    return pl.pallas_call(
        _fin_body,
        grid=(GR,),
        in_specs=[
            pl.BlockSpec((2, TR, H), lambda r: (0, r, 0)),
            pl.BlockSpec((TR, 1), lambda r: (r, 0)),
            pl.BlockSpec((1, D_HID), lambda r: (0, 0)),
            pl.BlockSpec((D_HID, D_HID), lambda r: (0, 0)),
            pl.BlockSpec((1, D_HID), lambda r: (0, 0)),
            pl.BlockSpec((D_HID, D_OUT), lambda r: (0, 0)),
            pl.BlockSpec((1, D_OUT), lambda r: (0, 0)),
        ],
        out_specs=pl.BlockSpec((TR, D_OUT), lambda r: (r, 0)),
        out_shape=jax.ShapeDtypeStruct((N, D_OUT), jnp.float32),
    )(agg, dinv, b3, Wo1, bo1, Wo2, bo2)


@jax.jit
def _run(x, src, dst, W1, b1, W2, b2, W3, b3, Wo1, bo1, Wo2, bo2):
    src_t = src.reshape(NT, EPT)
    src_t = jnp.pad(src_t, ((0, 0), (0, NCH * LANE - EPT))
                    ).reshape(NT, NCH, 1, LANE)
    dst_t = dst.reshape(NT, EPT)
    dst_t = jnp.pad(dst_t, ((0, 0), (0, NCH * LANE - EPT)),
                    constant_values=N).reshape(NT, NCH, 1, LANE)
    sd = jnp.concatenate([src_t, dst_t], axis=2)
    off = jnp.array([[Np], [0]], jnp.int32)
    sd2 = jnp.concatenate([sd[None], sd[None] + off], axis=0)
    sd2 = sd2.reshape(2 * NT, NCH, 2, LANE)
    ones_tab = jnp.ones((2 * Np, H), jnp.float32)
    degt = _agg(ones_tab, sd2)

    hp, dinv = _layer1(x, W1, degt)
    agg = _agg(hp.reshape(2 * Np, H), sd2).reshape(2, Np, H)

    hp = _mid(agg, dinv, b1.reshape(1, D_HID), W2)
    agg = _agg(hp.reshape(2 * Np, H), sd2).reshape(2, Np, H)

    hp = _mid(agg, dinv, b2.reshape(1, D_HID), W3)
    agg = _agg(hp.reshape(2 * Np, H), sd2).reshape(2, Np, H)

    return _final(agg, dinv, b3.reshape(1, D_HID), Wo1,
                  bo1.reshape(1, D_HID), Wo2, bo2.reshape(1, D_OUT))


def kernel(x, edge_index, W1, b1, W2, b2, W3, b3, Wo1, bo1, Wo2, bo2):
    src = edge_index[0].astype(jnp.int32)
    dst = edge_index[1].astype(jnp.int32)
    return _run(x, src, dst, W1, b1, W2, b2, W3, b3, Wo1, bo1, Wo2, bo2)


# speedup vs baseline: 10.0918x; 1.2529x over previous
import functools

import jax
import jax.numpy as jnp
from jax import lax
from jax.experimental import pallas as pl
from jax.experimental.pallas import tpu as pltpu
from jax.experimental.pallas import tpu_sc as plsc

N = 10000
E = 320000
D_IN = 128
D_HID = 256
D_OUT = 128
H = 128
LANE = 128
NT = 16
Np = 10240
RPT = Np // NT
EPT = E // NT
NCH = 2 * (-(-EPT // (2 * LANE)))
EPT_D = E // (2 * NT)
NCH_D = 2 * (-(-EPT_D // (2 * LANE)))
TR = 1000
GR = N // TR

_mesh = plsc.VectorSubcoreMesh(core_axis_name="c", subcore_axis_name="s")


@functools.partial(
    pl.kernel,
    out_type=jax.ShapeDtypeStruct((2 * Np, H), jnp.float32),
    mesh=_mesh,
    scratch_types=[
        pltpu.VMEM((2, LANE), jnp.int32),
        pltpu.VMEM((2, LANE), jnp.int32),
        pltpu.VMEM((LANE, H), jnp.float32),
        pltpu.VMEM((LANE, H), jnp.float32),
        pltpu.VMEM_SHARED((Np, H), jnp.float32),
        pltpu.SemaphoreType.DMA,
        pltpu.SemaphoreType.DMA,
    ],
)
def _agg(hp_hbm, sd_hbm, out_hbm, sd0, sd1, rows0, rows1, acc_sh, g0, g1):
    c = lax.axis_index("c")
    s = lax.axis_index("s")
    wid = c * NT + s
    base = c * Np + s * RPT
    pltpu.sync_copy(hp_hbm.at[pl.ds(base, RPT)], acc_sh.at[pl.ds(s * RPT, RPT)])
    plsc.subcore_barrier()

    pltpu.sync_copy(sd_hbm.at[wid].at[0], sd0)
    pltpu.async_copy(hp_hbm.at[sd0.at[0]], rows0, g0)

    def pair(i, carry):
        j = 2 * i
        pltpu.make_async_copy(hp_hbm.at[pl.ds(0, LANE)], rows0, g0).wait()
        pltpu.sync_copy(sd_hbm.at[wid].at[j + 1], sd1)
        pltpu.async_copy(hp_hbm.at[sd1.at[0]], rows1, g1)
        pltpu.sync_copy(rows0, acc_sh.at[sd0.at[1]], add=True)
        pltpu.make_async_copy(hp_hbm.at[pl.ds(0, LANE)], rows1, g1).wait()

        @pl.when(i < NCH // 2 - 1)
        def _():
            pltpu.sync_copy(sd_hbm.at[wid].at[j + 2], sd0)
            pltpu.async_copy(hp_hbm.at[sd0.at[0]], rows0, g0)

        pltpu.sync_copy(rows1, acc_sh.at[sd1.at[1]], add=True)
        return carry

    lax.fori_loop(0, NCH // 2, pair, 0)
    plsc.subcore_barrier()
    pltpu.sync_copy(acc_sh.at[pl.ds(s * RPT, RPT)], out_hbm.at[pl.ds(base, RPT)])


@functools.partial(
    pl.kernel,
    out_type=jax.ShapeDtypeStruct((2 * Np, H), jnp.float32),
    mesh=_mesh,
    scratch_types=[
        pltpu.VMEM((1, LANE), jnp.int32),
        pltpu.VMEM((1, LANE), jnp.int32),
        pltpu.VMEM((LANE, H), jnp.float32),
        pltpu.VMEM_SHARED((Np, H), jnp.float32),
    ],
)
def _degk(dst_hbm, ones_hbm, out_hbm, d0, d1, ones_v, acc_sh):
    c = lax.axis_index("c")
    s = lax.axis_index("s")
    wid = c * NT + s
    base = c * Np + s * RPT
    pltpu.sync_copy(ones_hbm, acc_sh.at[pl.ds(s * RPT, RPT)])
    pltpu.sync_copy(ones_hbm.at[pl.ds(0, LANE)], ones_v)
    plsc.subcore_barrier()

    pltpu.sync_copy(dst_hbm.at[wid].at[0], d0)

    def pair(i, carry):
        j = 2 * i
        pltpu.sync_copy(dst_hbm.at[wid].at[j + 1], d1)
        pltpu.sync_copy(ones_v, acc_sh.at[d0.at[0]], add=True)

        @pl.when(i < NCH_D // 2 - 1)
        def _():
            pltpu.sync_copy(dst_hbm.at[wid].at[j + 2], d0)

        pltpu.sync_copy(ones_v, acc_sh.at[d1.at[0]], add=True)
        return carry

    lax.fori_loop(0, NCH_D // 2, pair, 0)
    plsc.subcore_barrier()
    pltpu.sync_copy(acc_sh.at[pl.ds(s * RPT, RPT)], out_hbm.at[pl.ds(base, RPT)])


def _l1_body(x_ref, w_ref, degt_ref, hp_ref, dinv_ref):
    degt = degt_ref[...]
    deg = degt[0, :, :1] + degt[1, :, :1] - 1.0
    dinv = lax.rsqrt(deg)
    h = jnp.dot(x_ref[...], w_ref[...], preferred_element_type=jnp.float32)
    hp_ref[0] = h * dinv
    dinv_ref[...] = dinv


def _mid_body(agg_ref, dinv_ref, b_ref, w_ref, hp_ref):
    a = agg_ref[...]
    dinv = dinv_ref[...]
    b = b_ref[...]
    t0 = jnp.maximum(a[0] * dinv + b[:, :H], 0.0)
    t1 = jnp.maximum(a[1] * dinv + b[:, H:], 0.0)
    h = (jnp.dot(t0, w_ref[:H], preferred_element_type=jnp.float32)
         + jnp.dot(t1, w_ref[H:], preferred_element_type=jnp.float32))
    hp_ref[0] = h * dinv


def _fin_body(agg_ref, dinv_ref, b3_ref, wo1_ref, bo1_ref, wo2_ref, bo2_ref,
              out_ref):
    a = agg_ref[...]
    dinv = dinv_ref[...]
    b3 = b3_ref[...]
    t0 = jnp.maximum(a[0] * dinv + b3[:, :H], 0.0)
    t1 = jnp.maximum(a[1] * dinv + b3[:, H:], 0.0)
    u = (jnp.dot(t0, wo1_ref[:H], preferred_element_type=jnp.float32)
         + jnp.dot(t1, wo1_ref[H:], preferred_element_type=jnp.float32)
         + bo1_ref[...])
    out_ref[...] = (jnp.dot(u, wo2_ref[...], preferred_element_type=jnp.float32)
                    + bo2_ref[...])


def _layer1(x, W1, degt):
    return pl.pallas_call(
        _l1_body,
        grid=(GR, 2),
        in_specs=[
            pl.BlockSpec((TR, D_IN), lambda r, j: (r, 0)),
            pl.BlockSpec((D_IN, H), lambda r, j: (0, j)),
            pl.BlockSpec((2, TR, H), lambda r, j: (0, r, 0)),
        ],
        out_specs=[
            pl.BlockSpec((1, TR, H), lambda r, j: (j, r, 0)),
            pl.BlockSpec((TR, 1), lambda r, j: (r, 0)),
        ],
        out_shape=[
            jax.ShapeDtypeStruct((2, Np, H), jnp.float32),
            jax.ShapeDtypeStruct((N, 1), jnp.float32),
        ],
    )(x, W1, degt)


def _mid(agg, dinv, b, W):
    return pl.pallas_call(
        _mid_body,
        grid=(GR, 2),
        in_specs=[
            pl.BlockSpec((2, TR, H), lambda r, j: (0, r, 0)),
            pl.BlockSpec((TR, 1), lambda r, j: (r, 0)),
            pl.BlockSpec((1, D_HID), lambda r, j: (0, 0)),
            pl.BlockSpec((D_HID, H), lambda r, j: (0, j)),
        ],
        out_specs=pl.BlockSpec((1, TR, H), lambda r, j: (j, r, 0)),
        out_shape=jax.ShapeDtypeStruct((2, Np, H), jnp.float32),
    )(agg, dinv, b, W)


def _final(agg, dinv, b3, Wo1, bo1, Wo2, bo2):
    return pl.pallas_call(
        _fin_body,
        grid=(GR,),
        in_specs=[
            pl.BlockSpec((2, TR, H), lambda r: (0, r, 0)),
            pl.BlockSpec((TR, 1), lambda r: (r, 0)),
            pl.BlockSpec((1, D_HID), lambda r: (0, 0)),
            pl.BlockSpec((D_HID, D_HID), lambda r: (0, 0)),
            pl.BlockSpec((1, D_HID), lambda r: (0, 0)),
            pl.BlockSpec((D_HID, D_OUT), lambda r: (0, 0)),
            pl.BlockSpec((1, D_OUT), lambda r: (0, 0)),
        ],
        out_specs=pl.BlockSpec((TR, D_OUT), lambda r: (r, 0)),
        out_shape=jax.ShapeDtypeStruct((N, D_OUT), jnp.float32),
    )(agg, dinv, b3, Wo1, bo1, Wo2, bo2)


@jax.jit
def _run(x, src, dst, W1, b1, W2, b2, W3, b3, Wo1, bo1, Wo2, bo2):
    src_t = src.reshape(NT, EPT)
    src_t = jnp.pad(src_t, ((0, 0), (0, NCH * LANE - EPT))
                    ).reshape(NT, NCH, 1, LANE)
    dst_t = dst.reshape(NT, EPT)
    dst_t = jnp.pad(dst_t, ((0, 0), (0, NCH * LANE - EPT)),
                    constant_values=N).reshape(NT, NCH, 1, LANE)
    sd = jnp.concatenate([src_t, dst_t], axis=2)
    off = jnp.array([[Np], [0]], jnp.int32)
    sd2 = jnp.concatenate([sd[None], sd[None] + off], axis=0)
    sd2 = sd2.reshape(2 * NT, NCH, 2, LANE)
    dst_d = dst.reshape(2 * NT, EPT_D)
    dst_d = jnp.pad(dst_d, ((0, 0), (0, NCH_D * LANE - EPT_D)),
                    constant_values=N).reshape(2 * NT, NCH_D, 1, LANE)
    ones_tab = jnp.ones((RPT, H), jnp.float32)
    degt = _degk(dst_d, ones_tab).reshape(2, Np, H)

    hp, dinv = _layer1(x, W1, degt)
    agg = _agg(hp.reshape(2 * Np, H), sd2).reshape(2, Np, H)

    hp = _mid(agg, dinv, b1.reshape(1, D_HID), W2)
    agg = _agg(hp.reshape(2 * Np, H), sd2).reshape(2, Np, H)

    hp = _mid(agg, dinv, b2.reshape(1, D_HID), W3)
    agg = _agg(hp.reshape(2 * Np, H), sd2).reshape(2, Np, H)

    return _final(agg, dinv, b3.reshape(1, D_HID), Wo1,
                  bo1.reshape(1, D_HID), Wo2, bo2.reshape(1, D_OUT))


def kernel(x, edge_index, W1, b1, W2, b2, W3, b3, Wo1, bo1, Wo2, bo2):
    src = edge_index[0].astype(jnp.int32)
    dst = edge_index[1].astype(jnp.int32)
    return _run(x, src, dst, W1, b1, W2, b2, W3, b3, Wo1, bo1, Wo2, bo2)


# speedup vs baseline: 10.9997x; 1.0900x over previous
import functools

import jax
import jax.numpy as jnp
from jax import lax
from jax.experimental import pallas as pl
from jax.experimental.pallas import tpu as pltpu
from jax.experimental.pallas import tpu_sc as plsc

N = 10000
E = 320000
D_IN = 128
D_HID = 256
D_OUT = 128
H = 128
LANE = 128
NT = 16
Np = 10240
RPT = Np // NT
EPT = E // NT
NCH = 2 * (-(-EPT // (2 * LANE)))
EPT_D = E // (2 * NT)
NCH_D = 2 * (-(-EPT_D // (2 * LANE)))
TR = 1000
GR = N // TR

_mesh = plsc.VectorSubcoreMesh(core_axis_name="c", subcore_axis_name="s")


@functools.partial(
    pl.kernel,
    out_type=jax.ShapeDtypeStruct((2 * Np, H), jnp.float32),
    mesh=_mesh,
    scratch_types=[
        pltpu.VMEM((2, LANE), jnp.int32),
        pltpu.VMEM((2, LANE), jnp.int32),
        pltpu.VMEM((LANE, H), jnp.float32),
        pltpu.VMEM((LANE, H), jnp.float32),
        pltpu.VMEM_SHARED((Np, H), jnp.float32),
        pltpu.SemaphoreType.DMA,
        pltpu.SemaphoreType.DMA,
        pltpu.SemaphoreType.DMA,
        pltpu.SemaphoreType.DMA,
    ],
)
def _agg(hp_hbm, sd_hbm, out_hbm, sd0, sd1, rows0, rows1, acc_sh,
         g0, g1, s0, s1):
    c = lax.axis_index("c")
    s = lax.axis_index("s")
    wid = c * NT + s
    base = c * Np + s * RPT
    pltpu.sync_copy(hp_hbm.at[pl.ds(base, RPT)], acc_sh.at[pl.ds(s * RPT, RPT)])
    plsc.subcore_barrier()

    dummy = hp_hbm.at[pl.ds(0, LANE)]

    def wait(buf, sem):
        pltpu.make_async_copy(dummy, buf, sem).wait()

    pltpu.sync_copy(sd_hbm.at[wid].at[0], sd0)
    pltpu.async_copy(hp_hbm.at[sd0.at[0]], rows0, g0)
    pltpu.sync_copy(sd_hbm.at[wid].at[1], sd1)
    pltpu.async_copy(hp_hbm.at[sd1.at[0]], rows1, g1)

    def pair(i, carry):
        j = 2 * i
        wait(rows0, g0)
        pltpu.async_copy(rows0, acc_sh.at[sd0.at[1]], s0, add=True)
        wait(rows1, g1)

        @pl.when(i < NCH // 2 - 1)
        def _():
            wait(rows0, s0)
            pltpu.sync_copy(sd_hbm.at[wid].at[j + 2], sd0)
            pltpu.async_copy(hp_hbm.at[sd0.at[0]], rows0, g0)

        pltpu.async_copy(rows1, acc_sh.at[sd1.at[1]], s1, add=True)

        @pl.when(i < NCH // 2 - 1)
        def _():
            wait(rows1, s1)
            pltpu.sync_copy(sd_hbm.at[wid].at[j + 3], sd1)
            pltpu.async_copy(hp_hbm.at[sd1.at[0]], rows1, g1)

        return carry

    lax.fori_loop(0, NCH // 2, pair, 0)
    wait(rows0, s0)
    wait(rows1, s1)
    plsc.subcore_barrier()
    pltpu.sync_copy(acc_sh.at[pl.ds(s * RPT, RPT)], out_hbm.at[pl.ds(base, RPT)])


@functools.partial(
    pl.kernel,
    out_type=jax.ShapeDtypeStruct((2 * Np, H), jnp.float32),
    mesh=_mesh,
    scratch_types=[
        pltpu.VMEM((1, LANE), jnp.int32),
        pltpu.VMEM((1, LANE), jnp.int32),
        pltpu.VMEM((LANE, H), jnp.float32),
        pltpu.VMEM_SHARED((Np, H), jnp.float32),
    ],
)
def _degk(dst_hbm, ones_hbm, out_hbm, d0, d1, ones_v, acc_sh):
    c = lax.axis_index("c")
    s = lax.axis_index("s")
    wid = c * NT + s
    base = c * Np + s * RPT
    pltpu.sync_copy(ones_hbm, acc_sh.at[pl.ds(s * RPT, RPT)])
    pltpu.sync_copy(ones_hbm.at[pl.ds(0, LANE)], ones_v)
    plsc.subcore_barrier()

    pltpu.sync_copy(dst_hbm.at[wid].at[0], d0)

    def pair(i, carry):
        j = 2 * i
        pltpu.sync_copy(dst_hbm.at[wid].at[j + 1], d1)
        pltpu.sync_copy(ones_v, acc_sh.at[d0.at[0]], add=True)

        @pl.when(i < NCH_D // 2 - 1)
        def _():
            pltpu.sync_copy(dst_hbm.at[wid].at[j + 2], d0)

        pltpu.sync_copy(ones_v, acc_sh.at[d1.at[0]], add=True)
        return carry

    lax.fori_loop(0, NCH_D // 2, pair, 0)
    plsc.subcore_barrier()
    pltpu.sync_copy(acc_sh.at[pl.ds(s * RPT, RPT)], out_hbm.at[pl.ds(base, RPT)])


def _l1_body(x_ref, w_ref, degt_ref, hp_ref, dinv_ref):
    degt = degt_ref[...]
    deg = degt[0, :, :1] + degt[1, :, :1] - 1.0
    dinv = lax.rsqrt(deg)
    h = jnp.dot(x_ref[...], w_ref[...], preferred_element_type=jnp.float32)
    hp_ref[0] = h * dinv
    dinv_ref[...] = dinv


def _mid_body(agg_ref, dinv_ref, b_ref, w_ref, hp_ref):
    a = agg_ref[...]
    dinv = dinv_ref[...]
    b = b_ref[...]
    t0 = jnp.maximum(a[0] * dinv + b[:, :H], 0.0)
    t1 = jnp.maximum(a[1] * dinv + b[:, H:], 0.0)
    h = (jnp.dot(t0, w_ref[:H], preferred_element_type=jnp.float32)
         + jnp.dot(t1, w_ref[H:], preferred_element_type=jnp.float32))
    hp_ref[0] = h * dinv


def _fin_body(agg_ref, dinv_ref, b3_ref, wo1_ref, bo1_ref, wo2_ref, bo2_ref,
              out_ref):
    a = agg_ref[...]
    dinv = dinv_ref[...]
    b3 = b3_ref[...]
    t0 = jnp.maximum(a[0] * dinv + b3[:, :H], 0.0)
    t1 = jnp.maximum(a[1] * dinv + b3[:, H:], 0.0)
    u = (jnp.dot(t0, wo1_ref[:H], preferred_element_type=jnp.float32)
         + jnp.dot(t1, wo1_ref[H:], preferred_element_type=jnp.float32)
         + bo1_ref[...])
    out_ref[...] = (jnp.dot(u, wo2_ref[...], preferred_element_type=jnp.float32)
                    + bo2_ref[...])


def _layer1(x, W1, degt):
    return pl.pallas_call(
        _l1_body,
        grid=(GR, 2),
        in_specs=[
            pl.BlockSpec((TR, D_IN), lambda r, j: (r, 0)),
            pl.BlockSpec((D_IN, H), lambda r, j: (0, j)),
            pl.BlockSpec((2, TR, H), lambda r, j: (0, r, 0)),
        ],
        out_specs=[
            pl.BlockSpec((1, TR, H), lambda r, j: (j, r, 0)),
            pl.BlockSpec((TR, 1), lambda r, j: (r, 0)),
        ],
        out_shape=[
            jax.ShapeDtypeStruct((2, Np, H), jnp.float32),
            jax.ShapeDtypeStruct((N, 1), jnp.float32),
        ],
    )(x, W1, degt)


def _mid(agg, dinv, b, W):
    return pl.pallas_call(
        _mid_body,
        grid=(GR, 2),
        in_specs=[
            pl.BlockSpec((2, TR, H), lambda r, j: (0, r, 0)),
            pl.BlockSpec((TR, 1), lambda r, j: (r, 0)),
            pl.BlockSpec((1, D_HID), lambda r, j: (0, 0)),
            pl.BlockSpec((D_HID, H), lambda r, j: (0, j)),
        ],
        out_specs=pl.BlockSpec((1, TR, H), lambda r, j: (j, r, 0)),
        out_shape=jax.ShapeDtypeStruct((2, Np, H), jnp.float32),
    )(agg, dinv, b, W)


def _final(agg, dinv, b3, Wo1, bo1, Wo2, bo2):
    return pl.pallas_call(
        _fin_body,
        grid=(GR,),
        in_specs=[
            pl.BlockSpec((2, TR, H), lambda r: (0, r, 0)),
            pl.BlockSpec((TR, 1), lambda r: (r, 0)),
            pl.BlockSpec((1, D_HID), lambda r: (0, 0)),
            pl.BlockSpec((D_HID, D_HID), lambda r: (0, 0)),
            pl.BlockSpec((1, D_HID), lambda r: (0, 0)),
            pl.BlockSpec((D_HID, D_OUT), lambda r: (0, 0)),
            pl.BlockSpec((1, D_OUT), lambda r: (0, 0)),
        ],
        out_specs=pl.BlockSpec((TR, D_OUT), lambda r: (r, 0)),
        out_shape=jax.ShapeDtypeStruct((N, D_OUT), jnp.float32),
    )(agg, dinv, b3, Wo1, bo1, Wo2, bo2)


@jax.jit
def _run(x, src, dst, W1, b1, W2, b2, W3, b3, Wo1, bo1, Wo2, bo2):
    src_t = src.reshape(NT, EPT)
    src_t = jnp.pad(src_t, ((0, 0), (0, NCH * LANE - EPT))
                    ).reshape(NT, NCH, 1, LANE)
    dst_t = dst.reshape(NT, EPT)
    dst_t = jnp.pad(dst_t, ((0, 0), (0, NCH * LANE - EPT)),
                    constant_values=N).reshape(NT, NCH, 1, LANE)
    sd = jnp.concatenate([src_t, dst_t], axis=2)
    off = jnp.array([[Np], [0]], jnp.int32)
    sd2 = jnp.concatenate([sd[None], sd[None] + off], axis=0)
    sd2 = sd2.reshape(2 * NT, NCH, 2, LANE)
    dst_d = dst.reshape(2 * NT, EPT_D)
    dst_d = jnp.pad(dst_d, ((0, 0), (0, NCH_D * LANE - EPT_D)),
                    constant_values=N).reshape(2 * NT, NCH_D, 1, LANE)
    ones_tab = jnp.ones((RPT, H), jnp.float32)
    degt = _degk(dst_d, ones_tab).reshape(2, Np, H)

    hp, dinv = _layer1(x, W1, degt)
    agg = _agg(hp.reshape(2 * Np, H), sd2).reshape(2, Np, H)

    hp = _mid(agg, dinv, b1.reshape(1, D_HID), W2)
    agg = _agg(hp.reshape(2 * Np, H), sd2).reshape(2, Np, H)

    hp = _mid(agg, dinv, b2.reshape(1, D_HID), W3)
    agg = _agg(hp.reshape(2 * Np, H), sd2).reshape(2, Np, H)

    return _final(agg, dinv, b3.reshape(1, D_HID), Wo1,
                  bo1.reshape(1, D_HID), Wo2, bo2.reshape(1, D_OUT))


def kernel(x, edge_index, W1, b1, W2, b2, W3, b3, Wo1, bo1, Wo2, bo2):
    src = edge_index[0].astype(jnp.int32)
    dst = edge_index[1].astype(jnp.int32)
    return _run(x, src, dst, W1, b1, W2, b2, W3, b3, Wo1, bo1, Wo2, bo2)
